# baseline (speedup 1.0000x reference)
import dataclasses

import jax
import jax.numpy as jnp
from jax.experimental import pallas as pl
from jax.experimental.pallas import tpu as pltpu
from jax.experimental.pallas import tpu_sc as plsc

_N = 131072
_H = 32
_K = 512
_NB = 16384
_RC = 8

_NSC = _N // 2
_NW = 32
_PW = _NSC // _NW
_PT = 128
_NT = _PW // _PT


def _assign_body(cm2_ref, xt_ref, c2_ref, o_ref):
    cm2 = cm2_ref[...]
    dt = jax.lax.dot_general(
        cm2, xt_ref[...], (((1,), (0,)), ((), ())),
        preferred_element_type=jnp.float32) + c2_ref[...]

    best = dt[0:_RC, :]
    bi = jnp.zeros((_RC, _NB), jnp.float32)
    for c in range(1, _K // _RC):
        blk = dt[c * _RC:(c + 1) * _RC, :]
        take = blk < best
        best = jnp.where(take, blk, best)
        bi = jnp.where(take, jnp.float32(c), bi)
    srow = jax.lax.broadcasted_iota(jnp.int32, (_RC, _NB), 0).astype(jnp.float32)
    rowval = bi * jnp.float32(_RC) + srow
    m = jnp.min(best, axis=0, keepdims=True)
    idx = jnp.min(jnp.where(best <= m, rowval, jnp.float32(2 * _K)),
                  axis=0, keepdims=True)
    o_ref[...] = idx.astype(jnp.int32)


def _assign(xt, cm2, c2, n):
    grid = (n // _NB,)
    out = pl.pallas_call(
        _assign_body,
        grid=grid,
        in_specs=[
            pl.BlockSpec((_K, _H), lambda i: (0, 0)),
            pl.BlockSpec((_H, _NB), lambda i: (0, i)),
            pl.BlockSpec((_K, 1), lambda i: (0, 0)),
        ],
        out_specs=pl.BlockSpec((1, _NB), lambda i: (0, i)),
        out_shape=jax.ShapeDtypeStruct((1, n), jnp.int32),
        compiler_params=pltpu.CompilerParams(
            dimension_semantics=("arbitrary",)),
    )(cm2, xt, c2)
    return out.reshape(n, 1)


def _sc_transpose_body(x_hbm, o_hbm, tile_v, ot_v, sem):
    wid = jax.lax.axis_index("c") * 16 + jax.lax.axis_index("s")
    base = wid * _PW
    lane = jax.lax.iota(jnp.int32, 16)

    @pl.loop(0, _NT)
    def _tile(t):
        p0 = base + t * _PT
        pltpu.async_copy(x_hbm.at[pl.ds(p0, _PT), :], tile_v, sem).wait()

        @pl.loop(0, _H)
        def _row(h):
            col = jnp.full((16,), h, jnp.int32)
            for g in range(_PT // 16):
                rows = lane + (g * 16)
                v = plsc.load_gather(tile_v, [rows, col])
                ot_v[h, pl.ds(g * 16, 16)] = v

        pltpu.async_copy(ot_v, o_hbm.at[:, pl.ds(p0, _PT)], sem).wait()


def _sc_cp():
    cp = pltpu.CompilerParams()
    if "needs_layout_passes" in pltpu.CompilerParams.__dataclass_fields__:
        cp = dataclasses.replace(cp, needs_layout_passes=False)
    return cp


def _sc_transpose(x2):
    kern = pl.kernel(
        _sc_transpose_body,
        out_type=jax.ShapeDtypeStruct((_H, _NSC), jnp.float32),
        mesh=plsc.VectorSubcoreMesh(core_axis_name="c", subcore_axis_name="s"),
        compiler_params=_sc_cp(),
        scratch_types=[
            pltpu.VMEM((_PT, _H), jnp.float32),
            pltpu.VMEM((_H, _PT), jnp.float32),
            pltpu.SemaphoreType.DMA,
        ],
    )
    return kern(x2)


def kernel(x, cluster_centers):
    cm2 = -2.0 * cluster_centers
    c2 = jnp.sum(cluster_centers * cluster_centers, axis=1)[:, None]
    x1 = x[:_NSC]
    x2 = x[_NSC:]
    xt2 = _sc_transpose(x2)
    xt1 = x1.T
    o1 = _assign(xt1, cm2, c2, _NSC)
    o2 = _assign(xt2, cm2, c2, _NSC)
    return jnp.concatenate([o1, o2], axis=0)


# speedup vs baseline: 3.1799x; 3.1799x over previous
import jax
import jax.numpy as jnp
from jax.experimental import pallas as pl
from jax.experimental.pallas import tpu as pltpu

_N = 131072
_H = 32
_K = 512
_NB = 16384
_RC = 8


def _body(cm2_ref, xt_ref, c2_ref, o_ref):
    cm2 = cm2_ref[...]
    dt = jax.lax.dot_general(
        cm2, xt_ref[...], (((1,), (0,)), ((), ())),
        preferred_element_type=jnp.float32) + c2_ref[...]

    best = dt[0:_RC, :]
    bi = jnp.zeros((_RC, _NB), jnp.float32)
    for c in range(1, _K // _RC):
        blk = dt[c * _RC:(c + 1) * _RC, :]
        take = blk < best
        best = jnp.where(take, blk, best)
        bi = jnp.where(take, jnp.float32(c), bi)
    srow = jax.lax.broadcasted_iota(jnp.int32, (_RC, _NB), 0).astype(jnp.float32)
    rowval = bi * jnp.float32(_RC) + srow
    m = jnp.min(best, axis=0, keepdims=True)
    idx = jnp.min(jnp.where(best <= m, rowval, jnp.float32(2 * _K)),
                  axis=0, keepdims=True)
    o_ref[...] = idx.astype(jnp.int32)


def kernel(x, cluster_centers):
    cm2 = -2.0 * cluster_centers
    c2 = jnp.sum(cluster_centers * cluster_centers, axis=1)[:, None]
    xt = x.T
    grid = (_N // _NB,)
    out = pl.pallas_call(
        _body,
        grid=grid,
        in_specs=[
            pl.BlockSpec((_K, _H), lambda i: (0, 0)),
            pl.BlockSpec((_H, _NB), lambda i: (0, i)),
            pl.BlockSpec((_K, 1), lambda i: (0, 0)),
        ],
        out_specs=pl.BlockSpec((1, _NB), lambda i: (0, i)),
        out_shape=jax.ShapeDtypeStruct((1, _N), jnp.int32),
        compiler_params=pltpu.CompilerParams(
            dimension_semantics=("arbitrary",)),
    )(cm2, xt, c2)
    return out.reshape(_N, 1)
